# baseline (speedup 1.0000x reference)
import jax
import jax.numpy as jnp
from jax.experimental import pallas as pl
from jax.experimental.pallas import tpu as pltpu

_N, _CIN, _H, _W = 32, 64, 56, 56
_WP = 64
_HO, _WO = 28, 28
_WOP = 32
_CH = 32
_CL = 32
_SROWS = 60
_TROWS = 32


def _octave_kernel(x_ref, w_h_ref, w_l_ref, o_h_ref, o_l_ref, s_ref, t_ref):
    s_ref[...] = jnp.zeros_like(s_ref)
    t_ref[...] = jnp.zeros_like(t_ref)
    s_ref[pl.ds(2, _H), pl.ds(0, _W), :] = x_ref[...].astype(jnp.bfloat16)

    pooled = (x_ref[pl.ds(0, _HO, stride=2), pl.ds(0, _WO, stride=2), :]
              + x_ref[pl.ds(0, _HO, stride=2), pl.ds(1, _WO, stride=2), :]
              + x_ref[pl.ds(1, _HO, stride=2), pl.ds(0, _WO, stride=2), :]
              + x_ref[pl.ds(1, _HO, stride=2), pl.ds(1, _WO, stride=2), :])
    t_ref[pl.ds(2, _HO), pl.ds(0, _WO), :] = pooled.astype(jnp.bfloat16)

    def conv3x3(flat, wp, m):
        taps = [flat[(kh + 1) * wp + kw - 1:(kh + 1) * wp + kw - 1 + m, :]
                for kh in range(3) for kw in range(3)]
        return jnp.concatenate(taps, axis=-1)

    cols_h = conv3x3(s_ref[...].reshape(_SROWS * _WP, _CIN), _WP, _H * _WP)
    o_h_ref[...] = jnp.dot(cols_h, w_h_ref[...],
                           preferred_element_type=jnp.float32)

    cols_l = conv3x3(t_ref[...].reshape(_TROWS * _WOP, _CIN), _WOP, _HO * _WOP)
    o_l_ref[...] = jnp.dot(cols_l, w_l_ref[...],
                           preferred_element_type=jnp.float32)


def _pack_weight(w_oihw, scale=None):
    o, i, kh, kw = w_oihw.shape
    w = jnp.transpose(w_oihw, (2, 3, 1, 0)).reshape(kh * kw * i, o)
    if scale is not None:
        w = w * scale
    return w.astype(jnp.bfloat16)


def kernel(x_nchw, w_h2h_oihw, w_h2l_oihw):
    n = x_nchw.shape[0]
    x_nhwc = jnp.transpose(x_nchw, (0, 2, 3, 1))
    w_h = _pack_weight(w_h2h_oihw)
    w_l = _pack_weight(w_h2l_oihw, scale=0.25)

    mh = _H * _WP
    ml = _HO * _WOP

    out_h, out_l = pl.pallas_call(
        _octave_kernel,
        out_shape=(jax.ShapeDtypeStruct((n, mh, _CH), jnp.float32),
                   jax.ShapeDtypeStruct((n, ml, _CL), jnp.float32)),
        grid_spec=pltpu.PrefetchScalarGridSpec(
            num_scalar_prefetch=0,
            grid=(n,),
            in_specs=[
                pl.BlockSpec((None, _H, _W, _CIN), lambda i: (i, 0, 0, 0)),
                pl.BlockSpec((9 * _CIN, _CH), lambda i: (0, 0)),
                pl.BlockSpec((9 * _CIN, _CL), lambda i: (0, 0)),
            ],
            out_specs=[
                pl.BlockSpec((None, mh, _CH), lambda i: (i, 0, 0)),
                pl.BlockSpec((None, ml, _CL), lambda i: (i, 0, 0)),
            ],
            scratch_shapes=[
                pltpu.VMEM((_SROWS, _WP, _CIN), jnp.bfloat16),
                pltpu.VMEM((_TROWS, _WOP, _CIN), jnp.bfloat16),
            ],
        ),
        compiler_params=pltpu.CompilerParams(
            dimension_semantics=("parallel",),
            vmem_limit_bytes=64 * 1024 * 1024,
        ),
    )(x_nhwc, w_h, w_l)

    x_h = jnp.transpose(out_h.reshape(n, _H, _WP, _CH)[:, :, :_W, :], (0, 3, 1, 2))
    x_l = jnp.transpose(out_l.reshape(n, _HO, _WOP, _CL)[:, :, :_WO, :], (0, 3, 1, 2))
    return x_h, x_l
